# baseline (speedup 1.0000x reference)
import functools

import jax
import jax.numpy as jnp
from jax import lax
from jax.experimental import pallas as pl
from jax.experimental.pallas import tpu as pltpu

G, S, D, E, TOP_K = 2, 2048, 2048, 8, 2
CAP = 256
C_OUT = CAP - 1
BS = 256
NS = S // BS


def _router_body(ec_ref, x_ref, w_ref, b_ref, j_ref, comb_ref, disp_ref,
                 carry_ref):
    sblk = pl.program_id(1)

    @pl.when(sblk == 0)
    def _():
        carry_ref[...] = jnp.zeros_like(carry_ref)

    xb = x_ref[0].astype(jnp.bfloat16)
    wb = w_ref[...]
    bb = b_ref[...]

    logits = jnp.dot(xb, wb, preferred_element_type=jnp.float32) + bb
    m = jnp.max(logits, axis=1, keepdims=True)
    u = jnp.exp(logits - m)
    q = u / jnp.sum(u, axis=1, keepdims=True)
    qi = lax.bitcast_convert_type(q, jnp.int32)
    p = lax.bitcast_convert_type(qi & jnp.int32(-65536), jnp.float32)

    iota_e = lax.broadcasted_iota(jnp.int32, (BS, E), 1)
    v1 = jnp.max(p, axis=1, keepdims=True)
    sel1 = ((p - v1) == 0).astype(jnp.int32)
    idx1 = jnp.min(iota_e * sel1 + E * (1 - sel1), axis=1, keepdims=True)
    neg_inf = jnp.array(-jnp.inf, dtype=p.dtype)
    is1 = (iota_e - idx1) == 0
    p2 = jnp.where(is1, neg_inf, p)
    v2 = jnp.max(p2, axis=1, keepdims=True)
    sel2 = ((p2 - v2) == 0).astype(jnp.int32)
    idx2 = jnp.min(iota_e * sel2 + E * (1 - sel2), axis=1, keepdims=True)
    is2 = (iota_e - idx2) == 0

    oh1 = is1.astype(jnp.float32)
    oh2 = is2.astype(jnp.float32)
    mask16 = jnp.concatenate([oh1, oh2], axis=1)
    r_i = lax.broadcasted_iota(jnp.int32, (BS, BS), 0)
    c_i = lax.broadcasted_iota(jnp.int32, (BS, BS), 1)
    tri = (c_i <= r_i).astype(jnp.float32)
    pos = jnp.dot(tri, mask16, preferred_element_type=jnp.float32)
    pos = pos + carry_ref[...]
    carry_ref[...] = pos[BS - 1 : BS, :]

    pos1 = jnp.sum(pos[:, :E] * oh1, axis=1, keepdims=True).astype(jnp.int32)
    pos2 = jnp.sum(pos[:, E:] * oh2, axis=1, keepdims=True).astype(jnp.int32)

    ec = ec_ref[0, 0]
    lim = jnp.minimum(jnp.int32(CAP), ec)
    ok1 = (pos1 <= C_OUT) & (pos1 < lim)
    ok2 = (pos2 <= C_OUT) & (pos2 < lim)
    t1 = jnp.where(ok1, idx1 * C_OUT + pos1 - 1, -1)
    t2 = jnp.where(ok2, idx2 * C_OUT + pos2 - 1, -1)

    t1r = jnp.transpose(t1, (1, 0)).reshape(1, 1, BS)
    t2r = jnp.transpose(t2, (1, 0)).reshape(1, 1, BS)
    g1r = jnp.transpose(v1, (1, 0)).reshape(1, 1, BS)
    g2r = jnp.transpose(v2, (1, 0)).reshape(1, 1, BS)
    j3 = j_ref[...].reshape(C_OUT, E, 1)
    cmp1 = (j3 - t1r) == 0
    cmp2 = (j3 - t2r) == 0
    zero = jnp.zeros((), jnp.float32)
    comb_ref[0] = jnp.where(cmp1, g1r, jnp.where(cmp2, g2r, zero))
    disp_ref[0] = (cmp1 | cmp2).astype(jnp.int8)


@functools.partial(jax.jit, static_argnames=())
def kernel(x, expert_capacity, W, b):
    wb = W.astype(jnp.bfloat16)
    bb = b.astype(jnp.bfloat16).astype(jnp.float32).reshape(1, E)
    ec = jnp.asarray(expert_capacity, jnp.int32).reshape(1, 1)
    jarr = (jnp.arange(E, dtype=jnp.int32)[None, :] * C_OUT
            + jnp.arange(C_OUT, dtype=jnp.int32)[:, None])

    grid = (G, NS)
    comb, disp = pl.pallas_call(
        _router_body,
        grid=grid,
        in_specs=[
            pl.BlockSpec(memory_space=pltpu.SMEM),
            pl.BlockSpec((1, BS, D), lambda g, s: (g, s, 0)),
            pl.BlockSpec((D, E), lambda g, s: (0, 0)),
            pl.BlockSpec((1, E), lambda g, s: (0, 0)),
            pl.BlockSpec((C_OUT, E), lambda g, s: (0, 0)),
        ],
        out_specs=[
            pl.BlockSpec((1, C_OUT, E, BS), lambda g, s: (g, 0, 0, s)),
            pl.BlockSpec((1, C_OUT, E, BS), lambda g, s: (g, 0, 0, s)),
        ],
        out_shape=[
            jax.ShapeDtypeStruct((G, C_OUT, E, S), jnp.float32),
            jax.ShapeDtypeStruct((G, C_OUT, E, S), jnp.int8),
        ],
        scratch_shapes=[pltpu.VMEM((1, 2 * E), jnp.float32)],
    )(ec, x, wb, bb, jarr)
    return (jnp.transpose(comb, (0, 3, 2, 1)),
            jnp.transpose(disp, (0, 3, 2, 1)).astype(jnp.bool_), 0.0)


# speedup vs baseline: 1.0647x; 1.0647x over previous
import functools

import jax
import jax.numpy as jnp
from jax import lax
from jax.experimental import pallas as pl
from jax.experimental.pallas import tpu as pltpu

G, S, D, E, TOP_K = 2, 2048, 2048, 8, 2
CAP = 256
C_OUT = CAP - 1
BS = 512
NS = S // BS


def _router_body(ec_ref, x_ref, w_ref, b_ref, j_ref, comb_ref, disp_ref,
                 carry_ref):
    sblk = pl.program_id(1)

    @pl.when(sblk == 0)
    def _():
        carry_ref[...] = jnp.zeros_like(carry_ref)

    xb = x_ref[0].astype(jnp.bfloat16)
    wb = w_ref[...]
    bb = b_ref[...]

    logits = jnp.dot(xb, wb, preferred_element_type=jnp.float32) + bb
    m = jnp.max(logits, axis=1, keepdims=True)
    u = jnp.exp(logits - m)
    q = u / jnp.sum(u, axis=1, keepdims=True)
    qi = lax.bitcast_convert_type(q, jnp.int32)
    p = lax.bitcast_convert_type(qi & jnp.int32(-65536), jnp.float32)

    iota_e = lax.broadcasted_iota(jnp.int32, (BS, E), 1)
    v1 = jnp.max(p, axis=1, keepdims=True)
    sel1 = ((p - v1) == 0).astype(jnp.int32)
    idx1 = jnp.min(iota_e * sel1 + E * (1 - sel1), axis=1, keepdims=True)
    neg_inf = jnp.array(-jnp.inf, dtype=p.dtype)
    is1 = (iota_e - idx1) == 0
    p2 = jnp.where(is1, neg_inf, p)
    v2 = jnp.max(p2, axis=1, keepdims=True)
    sel2 = ((p2 - v2) == 0).astype(jnp.int32)
    idx2 = jnp.min(iota_e * sel2 + E * (1 - sel2), axis=1, keepdims=True)
    is2 = (iota_e - idx2) == 0

    oh1 = is1.astype(jnp.float32)
    oh2 = is2.astype(jnp.float32)
    mask16 = jnp.concatenate([oh1, oh2], axis=1)
    r_i = lax.broadcasted_iota(jnp.int32, (BS, BS), 0)
    c_i = lax.broadcasted_iota(jnp.int32, (BS, BS), 1)
    tri = (c_i <= r_i).astype(jnp.float32)
    pos = jnp.dot(tri, mask16, preferred_element_type=jnp.float32)
    pos = pos + carry_ref[...]
    carry_ref[...] = pos[BS - 1 : BS, :]

    pos1 = jnp.sum(pos[:, :E] * oh1, axis=1, keepdims=True).astype(jnp.int32)
    pos2 = jnp.sum(pos[:, E:] * oh2, axis=1, keepdims=True).astype(jnp.int32)

    ec = ec_ref[0, 0]
    lim = jnp.minimum(jnp.int32(CAP), ec)
    ok1 = (pos1 <= C_OUT) & (pos1 < lim)
    ok2 = (pos2 <= C_OUT) & (pos2 < lim)
    t1 = jnp.where(ok1, idx1 * C_OUT + pos1 - 1, -1)
    t2 = jnp.where(ok2, idx2 * C_OUT + pos2 - 1, -1)

    t1r = jnp.transpose(t1, (1, 0)).reshape(1, 1, BS)
    t2r = jnp.transpose(t2, (1, 0)).reshape(1, 1, BS)
    g1r = jnp.transpose(v1, (1, 0)).reshape(1, 1, BS)
    g2r = jnp.transpose(v2, (1, 0)).reshape(1, 1, BS)
    j3 = j_ref[...].reshape(C_OUT, E, 1)
    cmp1 = (j3 - t1r) == 0
    cmp2 = (j3 - t2r) == 0
    zero = jnp.zeros((), jnp.float32)
    comb_ref[0] = jnp.where(cmp1, g1r, jnp.where(cmp2, g2r, zero))
    disp_ref[0] = (cmp1 | cmp2).astype(jnp.int8)


@functools.partial(jax.jit, static_argnames=())
def kernel(x, expert_capacity, W, b):
    wb = W.astype(jnp.bfloat16)
    bb = b.astype(jnp.bfloat16).astype(jnp.float32).reshape(1, E)
    ec = jnp.asarray(expert_capacity, jnp.int32).reshape(1, 1)
    jarr = (jnp.arange(E, dtype=jnp.int32)[None, :] * C_OUT
            + jnp.arange(C_OUT, dtype=jnp.int32)[:, None])

    grid = (G, NS)
    comb, disp = pl.pallas_call(
        _router_body,
        grid=grid,
        in_specs=[
            pl.BlockSpec(memory_space=pltpu.SMEM),
            pl.BlockSpec((1, BS, D), lambda g, s: (g, s, 0)),
            pl.BlockSpec((D, E), lambda g, s: (0, 0)),
            pl.BlockSpec((1, E), lambda g, s: (0, 0)),
            pl.BlockSpec((C_OUT, E), lambda g, s: (0, 0)),
        ],
        out_specs=[
            pl.BlockSpec((1, C_OUT, E, BS), lambda g, s: (g, 0, 0, s)),
            pl.BlockSpec((1, C_OUT, E, BS), lambda g, s: (g, 0, 0, s)),
        ],
        out_shape=[
            jax.ShapeDtypeStruct((G, C_OUT, E, S), jnp.float32),
            jax.ShapeDtypeStruct((G, C_OUT, E, S), jnp.int8),
        ],
        scratch_shapes=[pltpu.VMEM((1, 2 * E), jnp.float32)],
    )(ec, x, wb, bb, jarr)
    return (jnp.transpose(comb, (0, 3, 2, 1)),
            jnp.transpose(disp, (0, 3, 2, 1)).astype(jnp.bool_), 0.0)
